# baseline (speedup 1.0000x reference)
import functools

import jax
import jax.numpy as jnp
from jax import lax
from jax.experimental import pallas as pl
from jax.experimental.pallas import tpu as pltpu
from jax.experimental.pallas import tpu_sc as plsc

_EMB_DIM = 64
_B = 4096
_BASKET = 20
_NC = 2
_NS = 16
_NW = _NC * _NS
_BPW = _B // _NW
_NBUF = 8
_LANES = 16
_DCOLS = _EMB_DIM // _LANES

_mesh = plsc.VectorSubcoreMesh(core_axis_name="c", subcore_axis_name="s")


@functools.partial(
    pl.kernel,
    mesh=_mesh,
    out_type=jax.ShapeDtypeStruct((_B, _EMB_DIM), jnp.float32),
    compiler_params=pltpu.CompilerParams(use_tc_tiling_on_sc=False),
    scratch_types=[
        pltpu.VMEM((_BPW, _BASKET), jnp.int32),
        pltpu.VMEM((_NBUF, _BASKET, _EMB_DIM), jnp.float32),
        pltpu.VMEM((_BPW, _EMB_DIM), jnp.float32),
        [pltpu.SemaphoreType.DMA] * _NBUF,
    ],
)
def _basket_sum(s_hbm, table_hbm, out_hbm, idx_v, rows_v, out_v, sems):
    wid = lax.axis_index("s") * _NC + lax.axis_index("c")
    ubase = wid * _BPW
    pltpu.sync_copy(
        s_hbm.at[pl.ds(ubase, _BPW), s_hbm.shape[1] - 1], idx_v
    )

    def gather(u, b):
        return pltpu.make_async_copy(
            table_hbm.at[idx_v.at[u]], rows_v.at[b], sems[b]
        )

    for b in range(_NBUF):
        gather(b, b).start()

    def outer(g, carry):
        for b in range(_NBUF):
            u = g * _NBUF + b
            gather(u, b).wait()
            for d in range(_DCOLS):
                acc = rows_v[b, 0, pl.ds(d * _LANES, _LANES)]
                for j in range(1, _BASKET):
                    acc = acc + rows_v[b, j, pl.ds(d * _LANES, _LANES)]
                out_v[u, pl.ds(d * _LANES, _LANES)] = acc

            @pl.when(u + _NBUF < _BPW)
            def _():
                gather(u + _NBUF, b).start()

        return carry

    lax.fori_loop(0, _BPW // _NBUF, outer, 0)
    pltpu.sync_copy(out_v, out_hbm.at[pl.ds(ubase, _BPW)])


def kernel(S, table):
    return _basket_sum(S.astype(jnp.int32), table)


# speedup vs baseline: 1.5220x; 1.5220x over previous
import functools

import jax
import jax.numpy as jnp
from jax import lax
from jax.experimental import pallas as pl
from jax.experimental.pallas import tpu as pltpu
from jax.experimental.pallas import tpu_sc as plsc

_EMB_DIM = 64
_B = 4096
_BASKET = 20
_PAD = 4
_STAGE = _BASKET + _PAD
_NC = 2
_NS = 16
_NW = _NC * _NS
_BPW = _B // _NW
_U = 4
_ROWS = _U * _BASKET
_STEPS = _BPW // _U
_NBUF = 4
_LANES = 16
_DCOLS = _EMB_DIM // _LANES

_mesh = plsc.VectorSubcoreMesh(core_axis_name="c", subcore_axis_name="s")


@functools.partial(
    pl.kernel,
    mesh=_mesh,
    out_type=jax.ShapeDtypeStruct((_B, _EMB_DIM), jnp.float32),
    compiler_params=pltpu.CompilerParams(
        use_tc_tiling_on_sc=False, needs_layout_passes=False
    ),
    scratch_types=[
        pltpu.VMEM((_BPW, _STAGE), jnp.int32),
        pltpu.VMEM((_BPW * _BASKET,), jnp.int32),
        pltpu.VMEM((_NBUF, _ROWS, _EMB_DIM), jnp.float32),
        pltpu.VMEM((_BPW, _EMB_DIM), jnp.float32),
        [pltpu.SemaphoreType.DMA] * _NBUF,
    ],
)
def _basket_sum(s_hbm, table_hbm, out_hbm, stage_v, idx_v, rows_v, out_v, sems):
    wid = lax.axis_index("s") * _NC + lax.axis_index("c")
    ubase = wid * _BPW
    lastcol = s_hbm.shape[1] - _STAGE
    pltpu.sync_copy(
        s_hbm.at[pl.ds(ubase, _BPW), pl.ds(lastcol, _STAGE)], stage_v
    )

    io = lax.iota(jnp.int32, _LANES)
    lo_mask = (io >= _PAD) & (io < 8)

    def repack(u, carry):
        base = u * _BASKET
        hi = stage_v[u, pl.ds(8, _LANES)]
        plsc.store_scatter(idx_v, [io + (base + _PAD)], hi)
        lo = stage_v[u, pl.ds(0, _LANES)]
        pos = jnp.where(lo_mask, io - _PAD + base, base)
        plsc.store_scatter(idx_v, [pos], lo, mask=lo_mask)
        return carry

    lax.fori_loop(0, _BPW, repack, 0)

    def gather(s, b):
        return pltpu.make_async_copy(
            table_hbm.at[idx_v.at[pl.ds(s * _ROWS, _ROWS)]], rows_v.at[b], sems[b]
        )

    for b in range(_NBUF):
        gather(b, b).start()

    def outer(g, carry):
        for b in range(_NBUF):
            s = g * _NBUF + b
            gather(s, b).wait()
            for u in range(_U):
                for d in range(_DCOLS):
                    acc = rows_v[b, u * _BASKET, pl.ds(d * _LANES, _LANES)]
                    for j in range(1, _BASKET):
                        acc = acc + rows_v[
                            b, u * _BASKET + j, pl.ds(d * _LANES, _LANES)
                        ]
                    out_v[s * _U + u, pl.ds(d * _LANES, _LANES)] = acc

            @pl.when(s + _NBUF < _STEPS)
            def _():
                gather(s + _NBUF, b).start()

        return carry

    lax.fori_loop(0, _STEPS // _NBUF, outer, 0)
    pltpu.sync_copy(out_v, out_hbm.at[pl.ds(ubase, _BPW)])


def kernel(S, table):
    s2 = S.astype(jnp.int32).reshape(_B, S.shape[1] * S.shape[2])
    return _basket_sum(s2, table)
